# baseline (speedup 1.0000x reference)
import functools

import jax
import jax.numpy as jnp
from jax import lax
from jax.experimental import pallas as pl
from jax.experimental.pallas import tpu as pltpu
from jax.experimental.pallas import tpu_sc as plsc

B, L, H = 4, 2048, 4096
N_EVID = 128
TOP_K = 3

_CH = 128
_NSTEP = H // _CH

_NC, _NS = 2, 16
_NW = _NC * _NS
_ROWS = B * N_EVID
_RPW = _ROWS // _NW


def _sc_gather_evidence(table, idx):
    mesh = plsc.VectorSubcoreMesh(core_axis_name="c", subcore_axis_name="s")

    @functools.partial(
        pl.kernel,
        mesh=mesh,
        out_type=jax.ShapeDtypeStruct((_ROWS, H), jnp.float32),
        scratch_types=[
            pltpu.VMEM((_RPW,), jnp.int32),
            pltpu.VMEM((_RPW, H), jnp.float32),
            pltpu.SemaphoreType.DMA,
        ],
    )
    def k(table_hbm, idx_hbm, out_hbm, idx_v, rows_v, sem):
        wid = lax.axis_index("s") * _NC + lax.axis_index("c")
        base = wid * _RPW
        pltpu.sync_copy(idx_hbm.at[pl.ds(base, _RPW)], idx_v)
        pltpu.async_copy(table_hbm.at[idx_v], rows_v, sem).wait()
        pltpu.sync_copy(rows_v, out_hbm.at[pl.ds(base, _RPW)])

    return k(table, idx)


_NA = 8


def _head_body(qe_ref, txt_ref, wq_ref, qpart_ref):
    c = pl.program_id(0)

    @pl.when(c == 0)
    def _():
        qpart_ref[...] = jnp.zeros_like(qpart_ref)

    t = txt_ref[:, 0, :]
    w = 1.0 - t
    qe3 = qe_ref[...]
    pooled_c = lax.dot_general(
        w.reshape(B, 1, L), qe3, (((2,), (1,)), ((0,), (0,))),
        preferred_element_type=jnp.float32).reshape(B, _CH)
    qpart_ref[...] += lax.dot_general(
        pooled_c, wq_ref[...], (((1,), (1,)), ((), ())),
        preferred_element_type=jnp.float32)


def _head(qe, txt3, Wq):
    return pl.pallas_call(
        _head_body,
        grid=(_NA,),
        in_specs=[
            pl.BlockSpec((B, L, _CH), lambda c: (0, 0, c)),
            pl.BlockSpec((B, 1, L), lambda c: (0, 0, 0)),
            pl.BlockSpec((H, _CH), lambda c: (0, c)),
        ],
        out_specs=pl.BlockSpec((B, H), lambda c: (0, 0)),
        out_shape=jax.ShapeDtypeStruct((B, H), jnp.float32),
        compiler_params=pltpu.CompilerParams(
            dimension_semantics=("arbitrary",)),
    )(qe, txt3, Wq)


def _fused_body(qe_ref, txt_ref, img_ref, ev_ref, we_ref, wq_ref, bq_ref,
                be_ref, qpart_ref, scores_ref, emb_ref, proj_acc, qacc):
    c = pl.program_id(0)

    @pl.when(c == 0)
    def _():
        proj_acc[...] = jnp.zeros_like(proj_acc)
        qacc[...] = qpart_ref[...]

    nt = (((1,), (1,)), ((), ()))
    t = txt_ref[:, 0, :]
    w = 1.0 - t

    @pl.when(c >= _NA)
    def _():
        qe3 = qe_ref[...]
        pooled_c = lax.dot_general(
            w.reshape(B, 1, L), qe3, (((2,), (1,)), ((0,), (0,))),
            preferred_element_type=jnp.float32).reshape(B, _CH)
        qacc[...] += lax.dot_general(
            pooled_c, wq_ref[...], nt, preferred_element_type=jnp.float32)

    ev_blk = ev_ref[:, pl.ds(c * _CH, _CH)]
    proj_acc[...] += lax.dot_general(
        ev_blk, we_ref[...], nt, preferred_element_type=jnp.float32)

    @pl.when(c == _NSTEP - 1)
    def _():
        eps = 1e-8
        im = img_ref[:, 0, :]
        wcol = jnp.sum(w, axis=1, keepdims=True)
        icol = jnp.sum(im, axis=1, keepdims=True)
        tcol = jnp.sum(t, axis=1, keepdims=True)
        inv_w = 1.0 / (wcol + 1e-6)
        query = qacc[...] * inv_w + bq_ref[...]
        qn = jnp.maximum(
            jnp.sqrt(jnp.sum(query * query, axis=1, keepdims=True)), eps)
        noise = 2.0 - icol * (1.0 / L)
        att = 0.5 + 0.5 * tcol * (1.0 / L)
        scale = noise * att / qn

        proj = proj_acc[...] + be_ref[...]
        p3 = proj.reshape(B, N_EVID, H)
        pn2 = jnp.sum(p3 * p3, axis=2)
        qd = jnp.sum(p3 * query.reshape(B, 1, H), axis=2)
        pn = jnp.maximum(jnp.sqrt(pn2), eps)
        scores = qd / pn * scale

        lane = lax.broadcasted_iota(jnp.int32, (1, N_EVID), 1)
        for b in range(B):
            cur = scores[b:b + 1, :]
            out_row = jnp.zeros((1, N_EVID), jnp.float32)
            for k in range(TOP_K):
                m = jnp.max(cur)
                am = jnp.min(jnp.where(cur == m, lane, N_EVID))
                out_row = jnp.where(lane == k, m, out_row)
                cur = jnp.where(lane == am, -jnp.inf, cur)
                emb_ref[b, k] = ev_ref[pl.ds(b * N_EVID + am, 1), :]
            scores_ref[pl.ds(b, 1), :] = out_row


def _fused(qe, txt3, img3, evidence, We, Wq, bq2, be2, qpart):
    return pl.pallas_call(
        _fused_body,
        grid=(_NSTEP,),
        in_specs=[
            pl.BlockSpec((B, L, _CH), lambda c: (0, 0, jnp.maximum(c, _NA))),
            pl.BlockSpec((B, 1, L), lambda c: (0, 0, 0)),
            pl.BlockSpec((B, 1, L), lambda c: (0, 0, 0)),
            pl.BlockSpec((_ROWS, H), lambda c: (0, 0)),
            pl.BlockSpec((H, _CH), lambda c: (0, c)),
            pl.BlockSpec((H, _CH), lambda c: (0, jnp.maximum(c, _NA))),
            pl.BlockSpec((1, H), lambda c: (0, 0)),
            pl.BlockSpec((1, H), lambda c: (0, 0)),
            pl.BlockSpec((B, H), lambda c: (0, 0)),
        ],
        out_specs=[
            pl.BlockSpec((B, N_EVID), lambda c: (0, 0)),
            pl.BlockSpec((B, TOP_K, 1, H), lambda c: (0, 0, 0, 0)),
        ],
        out_shape=[
            jax.ShapeDtypeStruct((B, N_EVID), jnp.float32),
            jax.ShapeDtypeStruct((B, TOP_K, 1, H), jnp.float32),
        ],
        scratch_shapes=[
            pltpu.VMEM((_ROWS, H), jnp.float32),
            pltpu.VMEM((B, H), jnp.float32),
        ],
        compiler_params=pltpu.CompilerParams(
            dimension_semantics=("arbitrary",)),
    )(qe, txt3, img3, evidence, We, Wq, bq2, be2, qpart)


def kernel(question_embeddings, evidence_tokens, img_conf, txt_conf,
           emb_table, Wq, bq, We, be):
    tokens = evidence_tokens.reshape(-1).astype(jnp.int32)
    evidence = _sc_gather_evidence(emb_table, tokens)

    txt3 = txt_conf.reshape(B, 1, L)
    img3 = img_conf.reshape(B, 1, L)
    qpart = _head(question_embeddings, txt3, Wq)
    scores128, topk_emb = _fused(question_embeddings, txt3, img3, evidence,
                                 We, Wq, bq.reshape(1, H), be.reshape(1, H),
                                 qpart)
    topk_scores = scores128[:, :TOP_K]
    return (topk_emb, topk_scores)


# speedup vs baseline: 1.0472x; 1.0472x over previous
import functools

import jax
import jax.numpy as jnp
from jax import lax
from jax.experimental import pallas as pl
from jax.experimental.pallas import tpu as pltpu
from jax.experimental.pallas import tpu_sc as plsc

B, L, H = 4, 2048, 4096
N_EVID = 128
TOP_K = 3

_CH = 256
_NSTEP = H // _CH

_NC, _NS = 2, 16
_NW = _NC * _NS
_ROWS = B * N_EVID
_RPW = _ROWS // _NW


def _sc_gather_evidence(table, idx):
    mesh = plsc.VectorSubcoreMesh(core_axis_name="c", subcore_axis_name="s")

    @functools.partial(
        pl.kernel,
        mesh=mesh,
        out_type=jax.ShapeDtypeStruct((_ROWS, H), jnp.float32),
        scratch_types=[
            pltpu.VMEM((_RPW,), jnp.int32),
            pltpu.VMEM((_RPW, H), jnp.float32),
            pltpu.SemaphoreType.DMA,
        ],
    )
    def k(table_hbm, idx_hbm, out_hbm, idx_v, rows_v, sem):
        wid = lax.axis_index("s") * _NC + lax.axis_index("c")
        base = wid * _RPW
        pltpu.sync_copy(idx_hbm.at[pl.ds(base, _RPW)], idx_v)
        pltpu.async_copy(table_hbm.at[idx_v], rows_v, sem).wait()
        pltpu.sync_copy(rows_v, out_hbm.at[pl.ds(base, _RPW)])

    return k(table, idx)


_NA = 4


def _head_body(qe_ref, txt_ref, wq_ref, qpart_ref):
    c = pl.program_id(0)

    @pl.when(c == 0)
    def _():
        qpart_ref[...] = jnp.zeros_like(qpart_ref)

    t = txt_ref[:, 0, :]
    w = 1.0 - t
    qe3 = qe_ref[...]
    pooled_c = lax.dot_general(
        w.reshape(B, 1, L), qe3, (((2,), (1,)), ((0,), (0,))),
        preferred_element_type=jnp.float32).reshape(B, _CH)
    qpart_ref[...] += lax.dot_general(
        pooled_c, wq_ref[...], (((1,), (1,)), ((), ())),
        preferred_element_type=jnp.float32)


def _head(qe, txt3, Wq):
    return pl.pallas_call(
        _head_body,
        grid=(_NA,),
        in_specs=[
            pl.BlockSpec((B, L, _CH), lambda c: (0, 0, c)),
            pl.BlockSpec((B, 1, L), lambda c: (0, 0, 0)),
            pl.BlockSpec((H, _CH), lambda c: (0, c)),
        ],
        out_specs=pl.BlockSpec((B, H), lambda c: (0, 0)),
        out_shape=jax.ShapeDtypeStruct((B, H), jnp.float32),
        compiler_params=pltpu.CompilerParams(
            dimension_semantics=("arbitrary",)),
    )(qe, txt3, Wq)


def _fused_body(qe_ref, txt_ref, img_ref, ev_ref, we_ref, wq_ref, bq_ref,
                be_ref, qpart_ref, scores_ref, emb_ref, proj_acc, qacc):
    c = pl.program_id(0)

    @pl.when(c == 0)
    def _():
        proj_acc[...] = jnp.zeros_like(proj_acc)
        qacc[...] = qpart_ref[...]

    nt = (((1,), (1,)), ((), ()))
    t = txt_ref[:, 0, :]
    w = 1.0 - t

    @pl.when(c >= _NA)
    def _():
        qe3 = qe_ref[...]
        pooled_c = lax.dot_general(
            w.reshape(B, 1, L), qe3, (((2,), (1,)), ((0,), (0,))),
            preferred_element_type=jnp.float32).reshape(B, _CH)
        qacc[...] += lax.dot_general(
            pooled_c, wq_ref[...], nt, preferred_element_type=jnp.float32)

    ev_blk = ev_ref[:, pl.ds(c * _CH, _CH)]
    proj_acc[...] += lax.dot_general(
        ev_blk, we_ref[...], nt, preferred_element_type=jnp.float32)

    @pl.when(c == _NSTEP - 1)
    def _():
        eps = 1e-8
        im = img_ref[:, 0, :]
        wcol = jnp.sum(w, axis=1, keepdims=True)
        icol = jnp.sum(im, axis=1, keepdims=True)
        tcol = jnp.sum(t, axis=1, keepdims=True)
        inv_w = 1.0 / (wcol + 1e-6)
        query = qacc[...] * inv_w + bq_ref[...]
        qn = jnp.maximum(
            jnp.sqrt(jnp.sum(query * query, axis=1, keepdims=True)), eps)
        noise = 2.0 - icol * (1.0 / L)
        att = 0.5 + 0.5 * tcol * (1.0 / L)
        scale = noise * att / qn

        proj = proj_acc[...] + be_ref[...]
        p3 = proj.reshape(B, N_EVID, H)
        pn2 = jnp.sum(p3 * p3, axis=2)
        qd = jnp.sum(p3 * query.reshape(B, 1, H), axis=2)
        pn = jnp.maximum(jnp.sqrt(pn2), eps)
        scores = qd / pn * scale

        lane = lax.broadcasted_iota(jnp.int32, (1, N_EVID), 1)
        for b in range(B):
            cur = scores[b:b + 1, :]
            out_row = jnp.zeros((1, N_EVID), jnp.float32)
            for k in range(TOP_K):
                m = jnp.max(cur)
                am = jnp.min(jnp.where(cur == m, lane, N_EVID))
                out_row = jnp.where(lane == k, m, out_row)
                cur = jnp.where(lane == am, -jnp.inf, cur)
                emb_ref[b, k] = ev_ref[pl.ds(b * N_EVID + am, 1), :]
            scores_ref[pl.ds(b, 1), :] = out_row


def _fused(qe, txt3, img3, evidence, We, Wq, bq2, be2, qpart):
    return pl.pallas_call(
        _fused_body,
        grid=(_NSTEP,),
        in_specs=[
            pl.BlockSpec((B, L, _CH), lambda c: (0, 0, jnp.maximum(c, _NA))),
            pl.BlockSpec((B, 1, L), lambda c: (0, 0, 0)),
            pl.BlockSpec((B, 1, L), lambda c: (0, 0, 0)),
            pl.BlockSpec((_ROWS, H), lambda c: (0, 0)),
            pl.BlockSpec((H, _CH), lambda c: (0, c)),
            pl.BlockSpec((H, _CH), lambda c: (0, jnp.maximum(c, _NA))),
            pl.BlockSpec((1, H), lambda c: (0, 0)),
            pl.BlockSpec((1, H), lambda c: (0, 0)),
            pl.BlockSpec((B, H), lambda c: (0, 0)),
        ],
        out_specs=[
            pl.BlockSpec((B, N_EVID), lambda c: (0, 0)),
            pl.BlockSpec((B, TOP_K, 1, H), lambda c: (0, 0, 0, 0)),
        ],
        out_shape=[
            jax.ShapeDtypeStruct((B, N_EVID), jnp.float32),
            jax.ShapeDtypeStruct((B, TOP_K, 1, H), jnp.float32),
        ],
        scratch_shapes=[
            pltpu.VMEM((_ROWS, H), jnp.float32),
            pltpu.VMEM((B, H), jnp.float32),
        ],
        compiler_params=pltpu.CompilerParams(
            dimension_semantics=("arbitrary",)),
    )(qe, txt3, img3, evidence, We, Wq, bq2, be2, qpart)


def kernel(question_embeddings, evidence_tokens, img_conf, txt_conf,
           emb_table, Wq, bq, We, be):
    tokens = evidence_tokens.reshape(-1).astype(jnp.int32)
    evidence = _sc_gather_evidence(emb_table, tokens)

    txt3 = txt_conf.reshape(B, 1, L)
    img3 = img_conf.reshape(B, 1, L)
    qpart = _head(question_embeddings, txt3, Wq)
    scores128, topk_emb = _fused(question_embeddings, txt3, img3, evidence,
                                 We, Wq, bq.reshape(1, H), be.reshape(1, H),
                                 qpart)
    topk_scores = scores128[:, :TOP_K]
    return (topk_emb, topk_scores)


# speedup vs baseline: 1.0479x; 1.0007x over previous
import functools

import jax
import jax.numpy as jnp
from jax import lax
from jax.experimental import pallas as pl
from jax.experimental.pallas import tpu as pltpu
from jax.experimental.pallas import tpu_sc as plsc

B, L, H = 4, 2048, 4096
N_EVID = 128
TOP_K = 3

_CH = 256
_NSTEP = H // _CH

_NC, _NS = 2, 16
_NW = _NC * _NS
_ROWS = B * N_EVID
_RPW = _ROWS // _NW


def _sc_gather_evidence(table, idx):
    mesh = plsc.VectorSubcoreMesh(core_axis_name="c", subcore_axis_name="s")

    @functools.partial(
        pl.kernel,
        mesh=mesh,
        out_type=jax.ShapeDtypeStruct((_ROWS, H), jnp.float32),
        scratch_types=[
            pltpu.VMEM((_RPW,), jnp.int32),
            pltpu.VMEM((_RPW, H), jnp.float32),
            pltpu.SemaphoreType.DMA,
        ],
    )
    def k(table_hbm, idx_hbm, out_hbm, idx_v, rows_v, sem):
        wid = lax.axis_index("s") * _NC + lax.axis_index("c")
        base = wid * _RPW
        pltpu.sync_copy(idx_hbm.at[pl.ds(base, _RPW)], idx_v)
        pltpu.async_copy(table_hbm.at[idx_v], rows_v, sem).wait()
        pltpu.sync_copy(rows_v, out_hbm.at[pl.ds(base, _RPW)])

    return k(table, idx)


_NA = 3


def _head_body(qe_ref, txt_ref, wq_ref, qpart_ref):
    c = pl.program_id(0)

    @pl.when(c == 0)
    def _():
        qpart_ref[...] = jnp.zeros_like(qpart_ref)

    t = txt_ref[:, 0, :]
    w = 1.0 - t
    qe3 = qe_ref[...]
    pooled_c = lax.dot_general(
        w.reshape(B, 1, L), qe3, (((2,), (1,)), ((0,), (0,))),
        preferred_element_type=jnp.float32).reshape(B, _CH)
    qpart_ref[...] += lax.dot_general(
        pooled_c, wq_ref[...], (((1,), (1,)), ((), ())),
        preferred_element_type=jnp.float32)


def _head(qe, txt3, Wq):
    return pl.pallas_call(
        _head_body,
        grid=(_NA,),
        in_specs=[
            pl.BlockSpec((B, L, _CH), lambda c: (0, 0, c)),
            pl.BlockSpec((B, 1, L), lambda c: (0, 0, 0)),
            pl.BlockSpec((H, _CH), lambda c: (0, c)),
        ],
        out_specs=pl.BlockSpec((B, H), lambda c: (0, 0)),
        out_shape=jax.ShapeDtypeStruct((B, H), jnp.float32),
        compiler_params=pltpu.CompilerParams(
            dimension_semantics=("arbitrary",)),
    )(qe, txt3, Wq)


def _fused_body(qe_ref, txt_ref, img_ref, ev_ref, we_ref, wq_ref, bq_ref,
                be_ref, qpart_ref, scores_ref, emb_ref, proj_acc, qacc):
    c = pl.program_id(0)

    @pl.when(c == 0)
    def _():
        proj_acc[...] = jnp.zeros_like(proj_acc)
        qacc[...] = qpart_ref[...]

    nt = (((1,), (1,)), ((), ()))
    t = txt_ref[:, 0, :]
    w = 1.0 - t

    @pl.when(c >= _NA)
    def _():
        qe3 = qe_ref[...]
        pooled_c = lax.dot_general(
            w.reshape(B, 1, L), qe3, (((2,), (1,)), ((0,), (0,))),
            preferred_element_type=jnp.float32).reshape(B, _CH)
        qacc[...] += lax.dot_general(
            pooled_c, wq_ref[...], nt, preferred_element_type=jnp.float32)

    ev_blk = ev_ref[:, pl.ds(c * _CH, _CH)]
    proj_acc[...] += lax.dot_general(
        ev_blk, we_ref[...], nt, preferred_element_type=jnp.float32)

    @pl.when(c == _NSTEP - 1)
    def _():
        eps = 1e-8
        im = img_ref[:, 0, :]
        wcol = jnp.sum(w, axis=1, keepdims=True)
        icol = jnp.sum(im, axis=1, keepdims=True)
        tcol = jnp.sum(t, axis=1, keepdims=True)
        inv_w = 1.0 / (wcol + 1e-6)
        query = qacc[...] * inv_w + bq_ref[...]
        qn = jnp.maximum(
            jnp.sqrt(jnp.sum(query * query, axis=1, keepdims=True)), eps)
        noise = 2.0 - icol * (1.0 / L)
        att = 0.5 + 0.5 * tcol * (1.0 / L)
        scale = noise * att / qn

        proj = proj_acc[...] + be_ref[...]
        p3 = proj.reshape(B, N_EVID, H)
        pn2 = jnp.sum(p3 * p3, axis=2)
        qd = jnp.sum(p3 * query.reshape(B, 1, H), axis=2)
        pn = jnp.maximum(jnp.sqrt(pn2), eps)
        scores = qd / pn * scale

        lane = lax.broadcasted_iota(jnp.int32, (1, N_EVID), 1)
        for b in range(B):
            cur = scores[b:b + 1, :]
            out_row = jnp.zeros((1, N_EVID), jnp.float32)
            for k in range(TOP_K):
                m = jnp.max(cur)
                am = jnp.min(jnp.where(cur == m, lane, N_EVID))
                out_row = jnp.where(lane == k, m, out_row)
                cur = jnp.where(lane == am, -jnp.inf, cur)
                emb_ref[b, k] = ev_ref[pl.ds(b * N_EVID + am, 1), :]
            scores_ref[pl.ds(b, 1), :] = out_row


def _fused(qe, txt3, img3, evidence, We, Wq, bq2, be2, qpart):
    return pl.pallas_call(
        _fused_body,
        grid=(_NSTEP,),
        in_specs=[
            pl.BlockSpec((B, L, _CH), lambda c: (0, 0, jnp.maximum(c, _NA))),
            pl.BlockSpec((B, 1, L), lambda c: (0, 0, 0)),
            pl.BlockSpec((B, 1, L), lambda c: (0, 0, 0)),
            pl.BlockSpec((_ROWS, H), lambda c: (0, 0)),
            pl.BlockSpec((H, _CH), lambda c: (0, c)),
            pl.BlockSpec((H, _CH), lambda c: (0, jnp.maximum(c, _NA))),
            pl.BlockSpec((1, H), lambda c: (0, 0)),
            pl.BlockSpec((1, H), lambda c: (0, 0)),
            pl.BlockSpec((B, H), lambda c: (0, 0)),
        ],
        out_specs=[
            pl.BlockSpec((B, N_EVID), lambda c: (0, 0)),
            pl.BlockSpec((B, TOP_K, 1, H), lambda c: (0, 0, 0, 0)),
        ],
        out_shape=[
            jax.ShapeDtypeStruct((B, N_EVID), jnp.float32),
            jax.ShapeDtypeStruct((B, TOP_K, 1, H), jnp.float32),
        ],
        scratch_shapes=[
            pltpu.VMEM((_ROWS, H), jnp.float32),
            pltpu.VMEM((B, H), jnp.float32),
        ],
        compiler_params=pltpu.CompilerParams(
            dimension_semantics=("arbitrary",)),
    )(qe, txt3, img3, evidence, We, Wq, bq2, be2, qpart)


def kernel(question_embeddings, evidence_tokens, img_conf, txt_conf,
           emb_table, Wq, bq, We, be):
    tokens = evidence_tokens.reshape(-1).astype(jnp.int32)
    evidence = _sc_gather_evidence(emb_table, tokens)

    txt3 = txt_conf.reshape(B, 1, L)
    img3 = img_conf.reshape(B, 1, L)
    qpart = _head(question_embeddings, txt3, Wq)
    scores128, topk_emb = _fused(question_embeddings, txt3, img3, evidence,
                                 We, Wq, bq.reshape(1, H), be.reshape(1, H),
                                 qpart)
    topk_scores = scores128[:, :TOP_K]
    return (topk_emb, topk_scores)


# speedup vs baseline: 1.0616x; 1.0131x over previous
import functools

import jax
import jax.numpy as jnp
from jax import lax
from jax.experimental import pallas as pl
from jax.experimental.pallas import tpu as pltpu
from jax.experimental.pallas import tpu_sc as plsc

B, L, H = 4, 2048, 4096
N_EVID = 128
TOP_K = 3

_CH = 256
_NSTEP = H // _CH

_NC, _NS = 2, 16
_NW = _NC * _NS
_ROWS = B * N_EVID
_RPW = _ROWS // _NW


def _sc_gather_evidence(table, idx):
    mesh = plsc.VectorSubcoreMesh(core_axis_name="c", subcore_axis_name="s")

    @functools.partial(
        pl.kernel,
        mesh=mesh,
        out_type=jax.ShapeDtypeStruct((_ROWS, H), jnp.float32),
        scratch_types=[
            pltpu.VMEM((_RPW,), jnp.int32),
            pltpu.VMEM((_RPW, H), jnp.float32),
            pltpu.SemaphoreType.DMA,
        ],
    )
    def k(table_hbm, idx_hbm, out_hbm, idx_v, rows_v, sem):
        wid = lax.axis_index("s") * _NC + lax.axis_index("c")
        base = wid * _RPW
        pltpu.sync_copy(idx_hbm.at[pl.ds(base, _RPW)], idx_v)
        pltpu.async_copy(table_hbm.at[idx_v], rows_v, sem).wait()
        pltpu.sync_copy(rows_v, out_hbm.at[pl.ds(base, _RPW)])

    return k(table, idx)


_NA = 2


def _head_body(qe_ref, txt_ref, wq_ref, qpart_ref):
    c = pl.program_id(0)

    @pl.when(c == 0)
    def _():
        qpart_ref[...] = jnp.zeros_like(qpart_ref)

    t = txt_ref[:, 0, :]
    w = 1.0 - t
    qe3 = qe_ref[...]
    pooled_c = lax.dot_general(
        w.reshape(B, 1, L), qe3, (((2,), (1,)), ((0,), (0,))),
        preferred_element_type=jnp.float32).reshape(B, _CH)
    qpart_ref[...] += lax.dot_general(
        pooled_c, wq_ref[...], (((1,), (1,)), ((), ())),
        preferred_element_type=jnp.float32)


def _head(qe, txt3, Wq):
    return pl.pallas_call(
        _head_body,
        grid=(_NA,),
        in_specs=[
            pl.BlockSpec((B, L, _CH), lambda c: (0, 0, c)),
            pl.BlockSpec((B, 1, L), lambda c: (0, 0, 0)),
            pl.BlockSpec((H, _CH), lambda c: (0, c)),
        ],
        out_specs=pl.BlockSpec((B, H), lambda c: (0, 0)),
        out_shape=jax.ShapeDtypeStruct((B, H), jnp.float32),
        compiler_params=pltpu.CompilerParams(
            dimension_semantics=("arbitrary",)),
    )(qe, txt3, Wq)


def _fused_body(qe_ref, txt_ref, img_ref, ev_ref, we_ref, wq_ref, bq_ref,
                be_ref, qpart_ref, scores_ref, emb_ref, proj_acc, qacc):
    c = pl.program_id(0)

    @pl.when(c == 0)
    def _():
        proj_acc[...] = jnp.zeros_like(proj_acc)
        qacc[...] = qpart_ref[...]

    nt = (((1,), (1,)), ((), ()))
    t = txt_ref[:, 0, :]
    w = 1.0 - t

    @pl.when(c >= _NA)
    def _():
        qe3 = qe_ref[...]
        pooled_c = lax.dot_general(
            w.reshape(B, 1, L), qe3, (((2,), (1,)), ((0,), (0,))),
            preferred_element_type=jnp.float32).reshape(B, _CH)
        qacc[...] += lax.dot_general(
            pooled_c, wq_ref[...], nt, preferred_element_type=jnp.float32)

    ev_blk = ev_ref[:, pl.ds(c * _CH, _CH)]
    proj_acc[...] += lax.dot_general(
        ev_blk, we_ref[...], nt, preferred_element_type=jnp.float32)

    @pl.when(c == _NSTEP - 1)
    def _():
        eps = 1e-8
        im = img_ref[:, 0, :]
        wcol = jnp.sum(w, axis=1, keepdims=True)
        icol = jnp.sum(im, axis=1, keepdims=True)
        tcol = jnp.sum(t, axis=1, keepdims=True)
        inv_w = 1.0 / (wcol + 1e-6)
        query = qacc[...] * inv_w + bq_ref[...]
        qn = jnp.maximum(
            jnp.sqrt(jnp.sum(query * query, axis=1, keepdims=True)), eps)
        noise = 2.0 - icol * (1.0 / L)
        att = 0.5 + 0.5 * tcol * (1.0 / L)
        scale = noise * att / qn

        proj = proj_acc[...] + be_ref[...]
        p3 = proj.reshape(B, N_EVID, H)
        pn2 = jnp.sum(p3 * p3, axis=2)
        qd = jnp.sum(p3 * query.reshape(B, 1, H), axis=2)
        pn = jnp.maximum(jnp.sqrt(pn2), eps)
        scores = qd / pn * scale

        lane = lax.broadcasted_iota(jnp.int32, (1, N_EVID), 1)
        for b in range(B):
            cur = scores[b:b + 1, :]
            out_row = jnp.zeros((1, N_EVID), jnp.float32)
            for k in range(TOP_K):
                m = jnp.max(cur)
                am = jnp.min(jnp.where(cur == m, lane, N_EVID))
                out_row = jnp.where(lane == k, m, out_row)
                cur = jnp.where(lane == am, -jnp.inf, cur)
                emb_ref[b, k] = ev_ref[pl.ds(b * N_EVID + am, 1), :]
            scores_ref[pl.ds(b, 1), :] = out_row


def _fused(qe, txt3, img3, evidence, We, Wq, bq2, be2, qpart):
    return pl.pallas_call(
        _fused_body,
        grid=(_NSTEP,),
        in_specs=[
            pl.BlockSpec((B, L, _CH), lambda c: (0, 0, jnp.maximum(c, _NA))),
            pl.BlockSpec((B, 1, L), lambda c: (0, 0, 0)),
            pl.BlockSpec((B, 1, L), lambda c: (0, 0, 0)),
            pl.BlockSpec((_ROWS, H), lambda c: (0, 0)),
            pl.BlockSpec((H, _CH), lambda c: (0, c)),
            pl.BlockSpec((H, _CH), lambda c: (0, jnp.maximum(c, _NA))),
            pl.BlockSpec((1, H), lambda c: (0, 0)),
            pl.BlockSpec((1, H), lambda c: (0, 0)),
            pl.BlockSpec((B, H), lambda c: (0, 0)),
        ],
        out_specs=[
            pl.BlockSpec((B, N_EVID), lambda c: (0, 0)),
            pl.BlockSpec((B, TOP_K, 1, H), lambda c: (0, 0, 0, 0)),
        ],
        out_shape=[
            jax.ShapeDtypeStruct((B, N_EVID), jnp.float32),
            jax.ShapeDtypeStruct((B, TOP_K, 1, H), jnp.float32),
        ],
        scratch_shapes=[
            pltpu.VMEM((_ROWS, H), jnp.float32),
            pltpu.VMEM((B, H), jnp.float32),
        ],
        compiler_params=pltpu.CompilerParams(
            dimension_semantics=("arbitrary",)),
    )(qe, txt3, img3, evidence, We, Wq, bq2, be2, qpart)


def kernel(question_embeddings, evidence_tokens, img_conf, txt_conf,
           emb_table, Wq, bq, We, be):
    tokens = evidence_tokens.reshape(-1).astype(jnp.int32)
    evidence = _sc_gather_evidence(emb_table, tokens)

    txt3 = txt_conf.reshape(B, 1, L)
    img3 = img_conf.reshape(B, 1, L)
    qpart = _head(question_embeddings, txt3, Wq)
    scores128, topk_emb = _fused(question_embeddings, txt3, img3, evidence,
                                 We, Wq, bq.reshape(1, H), be.reshape(1, H),
                                 qpart)
    topk_scores = scores128[:, :TOP_K]
    return (topk_emb, topk_scores)


# speedup vs baseline: 1.0826x; 1.0198x over previous
import functools

import jax
import jax.numpy as jnp
from jax import lax
from jax.experimental import pallas as pl
from jax.experimental.pallas import tpu as pltpu
from jax.experimental.pallas import tpu_sc as plsc

B, L, H = 4, 2048, 4096
N_EVID = 128
TOP_K = 3

_CH = 256
_NSTEP = H // _CH

_NC, _NS = 2, 16
_NW = _NC * _NS
_ROWS = B * N_EVID
_RPW = _ROWS // _NW


def _sc_gather_evidence(table, idx):
    mesh = plsc.VectorSubcoreMesh(core_axis_name="c", subcore_axis_name="s")

    @functools.partial(
        pl.kernel,
        mesh=mesh,
        out_type=jax.ShapeDtypeStruct((_ROWS, H), jnp.float32),
        scratch_types=[
            pltpu.VMEM((_RPW,), jnp.int32),
            pltpu.VMEM((_RPW, H), jnp.float32),
            pltpu.SemaphoreType.DMA,
        ],
    )
    def k(table_hbm, idx_hbm, out_hbm, idx_v, rows_v, sem):
        wid = lax.axis_index("s") * _NC + lax.axis_index("c")
        base = wid * _RPW
        pltpu.sync_copy(idx_hbm.at[pl.ds(base, _RPW)], idx_v)
        pltpu.async_copy(table_hbm.at[idx_v], rows_v, sem).wait()
        pltpu.sync_copy(rows_v, out_hbm.at[pl.ds(base, _RPW)])

    return k(table, idx)


_NA = 1


def _head_body(qe_ref, txt_ref, wq_ref, qpart_ref):
    c = pl.program_id(0)

    @pl.when(c == 0)
    def _():
        qpart_ref[...] = jnp.zeros_like(qpart_ref)

    t = txt_ref[:, 0, :]
    w = 1.0 - t
    qe3 = qe_ref[...]
    pooled_c = lax.dot_general(
        w.reshape(B, 1, L), qe3, (((2,), (1,)), ((0,), (0,))),
        preferred_element_type=jnp.float32).reshape(B, _CH)
    qpart_ref[...] += lax.dot_general(
        pooled_c, wq_ref[...], (((1,), (1,)), ((), ())),
        preferred_element_type=jnp.float32)


def _head(qe, txt3, Wq):
    return pl.pallas_call(
        _head_body,
        grid=(_NA,),
        in_specs=[
            pl.BlockSpec((B, L, _CH), lambda c: (0, 0, c)),
            pl.BlockSpec((B, 1, L), lambda c: (0, 0, 0)),
            pl.BlockSpec((H, _CH), lambda c: (0, c)),
        ],
        out_specs=pl.BlockSpec((B, H), lambda c: (0, 0)),
        out_shape=jax.ShapeDtypeStruct((B, H), jnp.float32),
        compiler_params=pltpu.CompilerParams(
            dimension_semantics=("arbitrary",)),
    )(qe, txt3, Wq)


def _fused_body(qe_ref, txt_ref, img_ref, ev_ref, we_ref, wq_ref, bq_ref,
                be_ref, qpart_ref, scores_ref, emb_ref, proj_acc, qacc):
    c = pl.program_id(0)

    @pl.when(c == 0)
    def _():
        proj_acc[...] = jnp.zeros_like(proj_acc)
        qacc[...] = qpart_ref[...]

    nt = (((1,), (1,)), ((), ()))
    t = txt_ref[:, 0, :]
    w = 1.0 - t

    @pl.when(c >= _NA)
    def _():
        qe3 = qe_ref[...]
        pooled_c = lax.dot_general(
            w.reshape(B, 1, L), qe3, (((2,), (1,)), ((0,), (0,))),
            preferred_element_type=jnp.float32).reshape(B, _CH)
        qacc[...] += lax.dot_general(
            pooled_c, wq_ref[...], nt, preferred_element_type=jnp.float32)

    ev_blk = ev_ref[:, pl.ds(c * _CH, _CH)]
    proj_acc[...] += lax.dot_general(
        ev_blk, we_ref[...], nt, preferred_element_type=jnp.float32)

    @pl.when(c == _NSTEP - 1)
    def _():
        eps = 1e-8
        im = img_ref[:, 0, :]
        wcol = jnp.sum(w, axis=1, keepdims=True)
        icol = jnp.sum(im, axis=1, keepdims=True)
        tcol = jnp.sum(t, axis=1, keepdims=True)
        inv_w = 1.0 / (wcol + 1e-6)
        query = qacc[...] * inv_w + bq_ref[...]
        qn = jnp.maximum(
            jnp.sqrt(jnp.sum(query * query, axis=1, keepdims=True)), eps)
        noise = 2.0 - icol * (1.0 / L)
        att = 0.5 + 0.5 * tcol * (1.0 / L)
        scale = noise * att / qn

        proj = proj_acc[...] + be_ref[...]
        p3 = proj.reshape(B, N_EVID, H)
        pn2 = jnp.sum(p3 * p3, axis=2)
        qd = jnp.sum(p3 * query.reshape(B, 1, H), axis=2)
        pn = jnp.maximum(jnp.sqrt(pn2), eps)
        scores = qd / pn * scale

        lane = lax.broadcasted_iota(jnp.int32, (1, N_EVID), 1)
        for b in range(B):
            cur = scores[b:b + 1, :]
            out_row = jnp.zeros((1, N_EVID), jnp.float32)
            for k in range(TOP_K):
                m = jnp.max(cur)
                am = jnp.min(jnp.where(cur == m, lane, N_EVID))
                out_row = jnp.where(lane == k, m, out_row)
                cur = jnp.where(lane == am, -jnp.inf, cur)
                emb_ref[b, k] = ev_ref[pl.ds(b * N_EVID + am, 1), :]
            scores_ref[pl.ds(b, 1), :] = out_row


def _fused(qe, txt3, img3, evidence, We, Wq, bq2, be2, qpart):
    return pl.pallas_call(
        _fused_body,
        grid=(_NSTEP,),
        in_specs=[
            pl.BlockSpec((B, L, _CH), lambda c: (0, 0, jnp.maximum(c, _NA))),
            pl.BlockSpec((B, 1, L), lambda c: (0, 0, 0)),
            pl.BlockSpec((B, 1, L), lambda c: (0, 0, 0)),
            pl.BlockSpec((_ROWS, H), lambda c: (0, 0)),
            pl.BlockSpec((H, _CH), lambda c: (0, c)),
            pl.BlockSpec((H, _CH), lambda c: (0, jnp.maximum(c, _NA))),
            pl.BlockSpec((1, H), lambda c: (0, 0)),
            pl.BlockSpec((1, H), lambda c: (0, 0)),
            pl.BlockSpec((B, H), lambda c: (0, 0)),
        ],
        out_specs=[
            pl.BlockSpec((B, N_EVID), lambda c: (0, 0)),
            pl.BlockSpec((B, TOP_K, 1, H), lambda c: (0, 0, 0, 0)),
        ],
        out_shape=[
            jax.ShapeDtypeStruct((B, N_EVID), jnp.float32),
            jax.ShapeDtypeStruct((B, TOP_K, 1, H), jnp.float32),
        ],
        scratch_shapes=[
            pltpu.VMEM((_ROWS, H), jnp.float32),
            pltpu.VMEM((B, H), jnp.float32),
        ],
        compiler_params=pltpu.CompilerParams(
            dimension_semantics=("arbitrary",)),
    )(qe, txt3, img3, evidence, We, Wq, bq2, be2, qpart)


def kernel(question_embeddings, evidence_tokens, img_conf, txt_conf,
           emb_table, Wq, bq, We, be):
    tokens = evidence_tokens.reshape(-1).astype(jnp.int32)
    evidence = _sc_gather_evidence(emb_table, tokens)

    txt3 = txt_conf.reshape(B, 1, L)
    img3 = img_conf.reshape(B, 1, L)
    qpart = _head(question_embeddings, txt3, Wq)
    scores128, topk_emb = _fused(question_embeddings, txt3, img3, evidence,
                                 We, Wq, bq.reshape(1, H), be.reshape(1, H),
                                 qpart)
    topk_scores = scores128[:, :TOP_K]
    return (topk_emb, topk_scores)
